# baseline (device time: 58677 ns/iter reference)
import jax
import jax.numpy as jnp
from jax import lax
from jax.experimental import pallas as pl
from jax.experimental.pallas import tpu as pltpu

N_DEV = 4
B, SQ, DMODEL = 2, 256, 512
HQ, DH = 4, 64
HD = HQ * DH
SKV = 256
BLK = 64


def kernel(x, Wq, K_ext, V_ext, Wo):
    k3 = K_ext.reshape(B, SKV, 16 * DH)
    v3 = V_ext.reshape(B, SKV, 16 * DH)

    def body(x_ref, wq_ref, k_ref, v_ref, wo_ref, out_ref,
             k_loc, v_loc, p_ref, r_ref,
             bc_send, bc_recv, ar_send, ar_recv):
        my_i = lax.axis_index("i")
        is_src = my_i == 0

        barrier = pltpu.get_barrier_semaphore()
        for d in range(1, N_DEV):
            pl.semaphore_signal(
                barrier, inc=1,
                device_id=((my_i + d) % N_DEV,),
                device_id_type=pl.DeviceIdType.MESH,
            )
        pl.semaphore_wait(barrier, N_DEV - 1)

        bc_rdmas = []
        for j in range(1, N_DEV):
            kr = pltpu.make_async_remote_copy(
                src_ref=k_ref.at[:, :, pl.ds(HD * j, HD)],
                dst_ref=k_loc,
                send_sem=bc_send.at[0, j - 1],
                recv_sem=bc_recv.at[0],
                device_id=(j,),
                device_id_type=pl.DeviceIdType.MESH,
            )
            vr = pltpu.make_async_remote_copy(
                src_ref=v_ref.at[:, :, pl.ds(HD * j, HD)],
                dst_ref=v_loc,
                send_sem=bc_send.at[1, j - 1],
                recv_sem=bc_recv.at[1],
                device_id=(j,),
                device_id_type=pl.DeviceIdType.MESH,
            )
            bc_rdmas.append((kr, vr))

        @pl.when(is_src)
        def _():
            for kr, vr in bc_rdmas:
                kr.start()
                vr.start()
            k_loc[...] = k_ref[:, :, 0:HD]
            v_loc[...] = v_ref[:, :, 0:HD]

        wq16 = wq_ref[...].astype(jnp.bfloat16)
        q = [
            jax.lax.dot(
                x_ref[b].astype(jnp.bfloat16), wq16,
                preferred_element_type=jnp.float32,
            )
            for b in range(B)
        ]

        @pl.when(jnp.logical_not(is_src))
        def _():
            kr0, vr0 = bc_rdmas[0]
            kr0.wait_recv()
            vr0.wait_recv()

        qb = lax.broadcasted_iota(jnp.int32, (SQ, SKV), 0) // BLK
        kb = lax.broadcasted_iota(jnp.int32, (SQ, SKV), 1) // BLK
        mask = kb <= qb

        wo16 = wo_ref[...].astype(jnp.bfloat16)
        for b in range(B):
            ctx_h = []
            for h in range(HQ):
                qh = q[b][:, h * DH:(h + 1) * DH].astype(jnp.bfloat16)
                kh = k_loc[b, :, h * DH:(h + 1) * DH].astype(jnp.bfloat16)
                s = jax.lax.dot_general(
                    qh, kh, (((1,), (1,)), ((), ())),
                    preferred_element_type=jnp.float32,
                ) * 0.125
                s = jnp.where(mask, s, -1e9)
                m = jnp.max(s, axis=1, keepdims=True)
                w = jnp.exp(s - m)
                w = w / jnp.sum(w, axis=1, keepdims=True)
                vh = v_loc[b, :, h * DH:(h + 1) * DH].astype(jnp.bfloat16)
                ctx_h.append(
                    jax.lax.dot(
                        w.astype(jnp.bfloat16), vh,
                        preferred_element_type=jnp.float32,
                    )
                )
            ctx = jnp.concatenate(ctx_h, axis=1)
            p_ref[b] = jax.lax.dot(
                ctx.astype(jnp.bfloat16), wo16,
                preferred_element_type=jnp.float32,
            )

        ar_rdmas = []
        for d in range(1, N_DEV):
            r = pltpu.make_async_remote_copy(
                src_ref=p_ref,
                dst_ref=r_ref.at[d - 1],
                send_sem=ar_send.at[d - 1],
                recv_sem=ar_recv.at[d - 1],
                device_id=((my_i + d) % N_DEV,),
                device_id_type=pl.DeviceIdType.MESH,
            )
            r.start()
            ar_rdmas.append(r)

        for r in ar_rdmas:
            r.wait_recv()
        out_ref[...] = p_ref[...] + r_ref[0] + r_ref[1] + r_ref[2]

        for r in ar_rdmas:
            r.wait_send()

        @pl.when(is_src)
        def _():
            for kr, vr in bc_rdmas:
                kr.wait_send()
                vr.wait_send()

    return pl.pallas_call(
        body,
        out_shape=jax.ShapeDtypeStruct((B, SQ, DMODEL), jnp.float32),
        in_specs=[pl.BlockSpec(memory_space=pltpu.VMEM)] * 5,
        out_specs=pl.BlockSpec(memory_space=pltpu.VMEM),
        scratch_shapes=[
            pltpu.VMEM((B, SKV, HD), jnp.float32),
            pltpu.VMEM((B, SKV, HD), jnp.float32),
            pltpu.VMEM((B, SQ, DMODEL), jnp.float32),
            pltpu.VMEM((N_DEV - 1, B, SQ, DMODEL), jnp.float32),
            pltpu.SemaphoreType.DMA((2, N_DEV - 1)),
            pltpu.SemaphoreType.DMA((2,)),
            pltpu.SemaphoreType.DMA((N_DEV - 1,)),
            pltpu.SemaphoreType.DMA((N_DEV - 1,)),
        ],
        compiler_params=pltpu.CompilerParams(collective_id=0),
    )(x, Wq, k3, v3, Wo)


# device time: 32775 ns/iter; 1.7903x vs baseline; 1.7903x over previous
import jax
import jax.numpy as jnp
from jax import lax
from jax.experimental import pallas as pl
from jax.experimental.pallas import tpu as pltpu

N_DEV = 4
B, SQ, DMODEL = 2, 256, 512
HQ, DH = 4, 64
HD = HQ * DH
SKV = 256
BLK = 64
QCOL = DMODEL // N_DEV


def kernel(x, Wq, K_ext, V_ext, Wo):
    k3 = K_ext.reshape(B, SKV, 16 * DH)
    v3 = V_ext.reshape(B, SKV, 16 * DH)

    def body(x_ref, wq_ref, k_ref, v_ref, wo_ref, out_ref,
             k16, v16, k_loc, v_loc, p16, rs_buf, ag_full,
             bc_send, bc_recv, rs_send, rs_recv, ag_send, ag_recv):
        my_i = lax.axis_index("i")
        is_src = my_i == 0

        barrier = pltpu.get_barrier_semaphore()
        for d in range(1, N_DEV):
            pl.semaphore_signal(
                barrier, inc=1,
                device_id=((my_i + d) % N_DEV,),
                device_id_type=pl.DeviceIdType.MESH,
            )
        pl.semaphore_wait(barrier, N_DEV - 1)

        bc_rdmas = []
        for j in range(1, N_DEV):
            kr = pltpu.make_async_remote_copy(
                src_ref=k16.at[:, :, pl.ds(HD * j, HD)],
                dst_ref=k_loc,
                send_sem=bc_send.at[0, j - 1],
                recv_sem=bc_recv.at[0],
                device_id=(j,),
                device_id_type=pl.DeviceIdType.MESH,
            )
            vr = pltpu.make_async_remote_copy(
                src_ref=v16.at[:, :, pl.ds(HD * j, HD)],
                dst_ref=v_loc,
                send_sem=bc_send.at[1, j - 1],
                recv_sem=bc_recv.at[1],
                device_id=(j,),
                device_id_type=pl.DeviceIdType.MESH,
            )
            bc_rdmas.append((kr, vr))

        @pl.when(is_src)
        def _():
            k16[...] = k_ref[...].astype(jnp.bfloat16)
            for kr, _ in bc_rdmas:
                kr.start()
            v16[...] = v_ref[...].astype(jnp.bfloat16)
            for _, vr in bc_rdmas:
                vr.start()
            k_loc[...] = k16[:, :, 0:HD]
            v_loc[...] = v16[:, :, 0:HD]

        wq16 = wq_ref[...].astype(jnp.bfloat16)
        q = [
            jax.lax.dot(
                x_ref[b].astype(jnp.bfloat16), wq16,
                preferred_element_type=jnp.float32,
            )
            for b in range(B)
        ]

        @pl.when(jnp.logical_not(is_src))
        def _():
            kr0, vr0 = bc_rdmas[0]
            kr0.wait_recv()
            vr0.wait_recv()

        qb = lax.broadcasted_iota(jnp.int32, (SQ, SKV), 0) // BLK
        kb = lax.broadcasted_iota(jnp.int32, (SQ, SKV), 1) // BLK
        mask = kb <= qb

        wo16 = wo_ref[...].astype(jnp.bfloat16)
        for b in range(B):
            ctx_h = []
            for h in range(HQ):
                qh = q[b][:, h * DH:(h + 1) * DH].astype(jnp.bfloat16)
                kh = k_loc[b, :, h * DH:(h + 1) * DH]
                s = jax.lax.dot_general(
                    qh, kh, (((1,), (1,)), ((), ())),
                    preferred_element_type=jnp.float32,
                ) * 0.125
                s = jnp.where(mask, s, -1e9)
                m = jnp.max(s, axis=1, keepdims=True)
                w = jnp.exp(s - m)
                w = w / jnp.sum(w, axis=1, keepdims=True)
                ctx_h.append(
                    jax.lax.dot(
                        w.astype(jnp.bfloat16), v_loc[b, :, h * DH:(h + 1) * DH],
                        preferred_element_type=jnp.float32,
                    )
                )
            ctx = jnp.concatenate(ctx_h, axis=1)
            p16[b] = jax.lax.dot(
                ctx.astype(jnp.bfloat16), wo16,
                preferred_element_type=jnp.float32,
            ).astype(jnp.bfloat16)

        rs_rdmas = []
        for d in range(1, N_DEV):
            j = (my_i + d) % N_DEV
            r = pltpu.make_async_remote_copy(
                src_ref=p16.at[:, :, pl.ds(QCOL * j, QCOL)],
                dst_ref=rs_buf.at[d - 1],
                send_sem=rs_send.at[d - 1],
                recv_sem=rs_recv.at[d - 1],
                device_id=(j,),
                device_id_type=pl.DeviceIdType.MESH,
            )
            r.start()
            rs_rdmas.append(r)
        for r in rs_rdmas:
            r.wait_recv()

        q_red = p16[:, :, pl.ds(QCOL * my_i, QCOL)].astype(jnp.float32)
        for d in range(N_DEV - 1):
            q_red = q_red + rs_buf[d].astype(jnp.float32)
        ag_full[:, :, pl.ds(QCOL * my_i, QCOL)] = q_red.astype(jnp.bfloat16)

        ag_rdmas = []
        for d in range(1, N_DEV):
            j = (my_i + d) % N_DEV
            r = pltpu.make_async_remote_copy(
                src_ref=ag_full.at[:, :, pl.ds(QCOL * my_i, QCOL)],
                dst_ref=ag_full.at[:, :, pl.ds(QCOL * my_i, QCOL)],
                send_sem=ag_send.at[d - 1],
                recv_sem=ag_recv.at[d - 1],
                device_id=(j,),
                device_id_type=pl.DeviceIdType.MESH,
            )
            r.start()
            ag_rdmas.append(r)
        for r in ag_rdmas:
            r.wait_recv()

        out_ref[...] = ag_full[...].astype(jnp.float32)

        for r in rs_rdmas:
            r.wait_send()
        for r in ag_rdmas:
            r.wait_send()

        @pl.when(is_src)
        def _():
            for kr, vr in bc_rdmas:
                kr.wait_send()
                vr.wait_send()

    return pl.pallas_call(
        body,
        out_shape=jax.ShapeDtypeStruct((B, SQ, DMODEL), jnp.float32),
        in_specs=[pl.BlockSpec(memory_space=pltpu.VMEM)] * 5,
        out_specs=pl.BlockSpec(memory_space=pltpu.VMEM),
        scratch_shapes=[
            pltpu.VMEM((B, SKV, 16 * DH), jnp.bfloat16),
            pltpu.VMEM((B, SKV, 16 * DH), jnp.bfloat16),
            pltpu.VMEM((B, SKV, HD), jnp.bfloat16),
            pltpu.VMEM((B, SKV, HD), jnp.bfloat16),
            pltpu.VMEM((B, SQ, DMODEL), jnp.bfloat16),
            pltpu.VMEM((N_DEV - 1, B, SQ, QCOL), jnp.bfloat16),
            pltpu.VMEM((B, SQ, DMODEL), jnp.bfloat16),
            pltpu.SemaphoreType.DMA((2, N_DEV - 1)),
            pltpu.SemaphoreType.DMA((2,)),
            pltpu.SemaphoreType.DMA((N_DEV - 1,)),
            pltpu.SemaphoreType.DMA((N_DEV - 1,)),
            pltpu.SemaphoreType.DMA((N_DEV - 1,)),
            pltpu.SemaphoreType.DMA((N_DEV - 1,)),
        ],
        compiler_params=pltpu.CompilerParams(collective_id=0),
    )(x, Wq, k3, v3, Wo)


# device time: 31047 ns/iter; 1.8899x vs baseline; 1.0557x over previous
import jax
import jax.numpy as jnp
from jax import lax
from jax.experimental import pallas as pl
from jax.experimental.pallas import tpu as pltpu

N_DEV = 4
B, SQ, DMODEL = 2, 256, 512
HQ, DH = 4, 64
HD = HQ * DH
SKV = 256
BLK = 64
QCOL = DMODEL // N_DEV


def kernel(x, Wq, K_ext, V_ext, Wo):
    k3 = K_ext.reshape(B, SKV, 16 * DH)
    v3 = V_ext.reshape(B, SKV, 16 * DH)

    def body(x_ref, wq_ref, k_ref, v_ref, wo_ref, out_ref,
             k16, v16, k_loc, v_loc, p16, rs_buf, ag_full, wo16,
             bc_send, bc_recv, rs_send, rs_recv, ag_send, ag_recv):
        my_i = lax.axis_index("i")
        is_src = my_i == 0

        barrier = pltpu.get_barrier_semaphore()
        for d in range(1, N_DEV):
            pl.semaphore_signal(
                barrier, inc=1,
                device_id=((my_i + d) % N_DEV,),
                device_id_type=pl.DeviceIdType.MESH,
            )
        pl.semaphore_wait(barrier, N_DEV - 1)

        def bc_copy(ref16, loc_ref, kv, j):
            return pltpu.make_async_remote_copy(
                src_ref=ref16.at[:, :, pl.ds(HD * j, HD)],
                dst_ref=loc_ref,
                send_sem=bc_send.at[kv, j - 1],
                recv_sem=bc_recv.at[kv],
                device_id=(j,),
                device_id_type=pl.DeviceIdType.MESH,
            )

        bc_order = [2, 1, 3]
        k_rdmas = {j: bc_copy(k16, k_loc, 0, j) for j in bc_order}
        v_rdmas = {j: bc_copy(v16, v_loc, 1, j) for j in bc_order}

        @pl.when(is_src)
        def _():
            k16[...] = k_ref[...].astype(jnp.bfloat16)
            v16[...] = v_ref[...].astype(jnp.bfloat16)
            for j in bc_order:
                k_rdmas[j].start()
                v_rdmas[j].start()
            k_loc[...] = k16[:, :, 0:HD]
            v_loc[...] = v16[:, :, 0:HD]

        wq16 = wq_ref[...].astype(jnp.bfloat16)
        q = [
            jax.lax.dot(
                x_ref[b].astype(jnp.bfloat16), wq16,
                preferred_element_type=jnp.float32,
            )
            for b in range(B)
        ]

        @pl.when(jnp.logical_not(is_src))
        def _():
            k_rdmas[2].wait_recv()

        qb = lax.broadcasted_iota(jnp.int32, (SQ, SKV), 0) // BLK
        kb = lax.broadcasted_iota(jnp.int32, (SQ, SKV), 1) // BLK
        mask = kb <= qb

        ws = []
        for b in range(B):
            for h in range(HQ):
                qh = q[b][:, h * DH:(h + 1) * DH].astype(jnp.bfloat16)
                kh = k_loc[b, :, h * DH:(h + 1) * DH]
                s = jax.lax.dot_general(
                    qh, kh, (((1,), (1,)), ((), ())),
                    preferred_element_type=jnp.float32,
                ) * 0.125
                w = jnp.where(mask, jnp.exp(s), 0.0)
                w = (w / jnp.sum(w, axis=1, keepdims=True)).astype(jnp.bfloat16)
                ws.append(w)

        @pl.when(jnp.logical_not(is_src))
        def _():
            v_rdmas[2].wait_recv()

        ctx = []
        for b in range(B):
            ctx_h = [
                jax.lax.dot(
                    ws[b * HQ + h], v_loc[b, :, h * DH:(h + 1) * DH],
                    preferred_element_type=jnp.float32,
                )
                for h in range(HQ)
            ]
            ctx.append(
                jnp.concatenate(ctx_h, axis=1).astype(jnp.bfloat16)
            )

        wo16[...] = wo_ref[...].astype(jnp.bfloat16)
        rs_rdmas = []
        for d in range(1, N_DEV):
            j = (my_i + d) % N_DEV
            wo_q = wo16[:, pl.ds(QCOL * j, QCOL)]
            for b in range(B):
                p16[b, :, pl.ds(QCOL * j, QCOL)] = jax.lax.dot(
                    ctx[b], wo_q, preferred_element_type=jnp.float32,
                ).astype(jnp.bfloat16)
            r = pltpu.make_async_remote_copy(
                src_ref=p16.at[:, :, pl.ds(QCOL * j, QCOL)],
                dst_ref=rs_buf.at[d - 1],
                send_sem=rs_send.at[d - 1],
                recv_sem=rs_recv.at[d - 1],
                device_id=(j,),
                device_id_type=pl.DeviceIdType.MESH,
            )
            r.start()
            rs_rdmas.append(r)

        wo_mine = wo16[:, pl.ds(QCOL * my_i, QCOL)]
        q_red = jnp.stack(
            [
                jax.lax.dot(ctx[b], wo_mine, preferred_element_type=jnp.float32)
                for b in range(B)
            ],
            axis=0,
        )

        for r in rs_rdmas:
            r.wait_recv()
        for d in range(N_DEV - 1):
            q_red = q_red + rs_buf[d].astype(jnp.float32)
        ag_full[:, :, pl.ds(QCOL * my_i, QCOL)] = q_red.astype(jnp.bfloat16)

        ag_rdmas = []
        for d in range(1, N_DEV):
            j = (my_i + d) % N_DEV
            r = pltpu.make_async_remote_copy(
                src_ref=ag_full.at[:, :, pl.ds(QCOL * my_i, QCOL)],
                dst_ref=ag_full.at[:, :, pl.ds(QCOL * my_i, QCOL)],
                send_sem=ag_send.at[d - 1],
                recv_sem=ag_recv.at[d - 1],
                device_id=(j,),
                device_id_type=pl.DeviceIdType.MESH,
            )
            r.start()
            ag_rdmas.append(r)
        for r in ag_rdmas:
            r.wait_recv()

        out_ref[...] = ag_full[...].astype(jnp.float32)

        for r in rs_rdmas:
            r.wait_send()
        for r in ag_rdmas:
            r.wait_send()

        @pl.when(is_src)
        def _():
            for j in bc_order:
                k_rdmas[j].wait_send()
                v_rdmas[j].wait_send()

    return pl.pallas_call(
        body,
        out_shape=jax.ShapeDtypeStruct((B, SQ, DMODEL), jnp.float32),
        in_specs=[pl.BlockSpec(memory_space=pltpu.VMEM)] * 5,
        out_specs=pl.BlockSpec(memory_space=pltpu.VMEM),
        scratch_shapes=[
            pltpu.VMEM((B, SKV, 16 * DH), jnp.bfloat16),
            pltpu.VMEM((B, SKV, 16 * DH), jnp.bfloat16),
            pltpu.VMEM((B, SKV, HD), jnp.bfloat16),
            pltpu.VMEM((B, SKV, HD), jnp.bfloat16),
            pltpu.VMEM((B, SQ, DMODEL), jnp.bfloat16),
            pltpu.VMEM((N_DEV - 1, B, SQ, QCOL), jnp.bfloat16),
            pltpu.VMEM((B, SQ, DMODEL), jnp.bfloat16),
            pltpu.VMEM((HD, DMODEL), jnp.bfloat16),
            pltpu.SemaphoreType.DMA((2, N_DEV - 1)),
            pltpu.SemaphoreType.DMA((2,)),
            pltpu.SemaphoreType.DMA((N_DEV - 1,)),
            pltpu.SemaphoreType.DMA((N_DEV - 1,)),
            pltpu.SemaphoreType.DMA((N_DEV - 1,)),
            pltpu.SemaphoreType.DMA((N_DEV - 1,)),
        ],
        compiler_params=pltpu.CompilerParams(collective_id=0),
    )(x, Wq, k3, v3, Wo)


# device time: 28991 ns/iter; 2.0240x vs baseline; 1.0709x over previous
import jax
import jax.numpy as jnp
from jax import lax
from jax.experimental import pallas as pl
from jax.experimental.pallas import tpu as pltpu

N_DEV = 4
B, SQ, DMODEL = 2, 256, 512
HQ, DH = 4, 64
HD = HQ * DH
SKV = 256
BLK = 64
QCOL = DMODEL // N_DEV
CW = 2 * DH
NCH = HD // CW

BC_ORDER = [2, 1, 3]


def kernel(x, Wq, K_ext, V_ext, Wo):
    k3 = K_ext.reshape(B, SKV, 16 * DH)
    v3 = V_ext.reshape(B, SKV, 16 * DH)

    def body(x_ref, wq_ref, k_ref, v_ref, wo_ref, out_ref,
             k16, v16, k_loc, v_loc, p16, rs_buf, ag_full, wo16,
             bck_send, bck_recv, bcv_send, bcv_recv,
             rs_send, rs_recv, ag_send, ag_recv):
        my_i = lax.axis_index("i")
        is_src = my_i == 0
        not_src = jnp.logical_not(is_src)

        barrier = pltpu.get_barrier_semaphore()
        for d in range(1, N_DEV):
            pl.semaphore_signal(
                barrier, inc=1,
                device_id=((my_i + d) % N_DEV,),
                device_id_type=pl.DeviceIdType.MESH,
            )
        pl.semaphore_wait(barrier, N_DEV - 1)

        def k_copy(j, c):
            return pltpu.make_async_remote_copy(
                src_ref=k16.at[:, :, pl.ds(HD * j + CW * c, CW)],
                dst_ref=k_loc.at[:, :, pl.ds(CW * c, CW)],
                send_sem=bck_send.at[c, j - 1],
                recv_sem=bck_recv.at[c],
                device_id=(j,),
                device_id_type=pl.DeviceIdType.MESH,
            )

        def v_copy(j, b, c):
            return pltpu.make_async_remote_copy(
                src_ref=v16.at[b, :, pl.ds(HD * j + CW * c, CW)],
                dst_ref=v_loc.at[b, :, pl.ds(CW * c, CW)],
                send_sem=bcv_send.at[b, c, j - 1],
                recv_sem=bcv_recv.at[b, c],
                device_id=(j,),
                device_id_type=pl.DeviceIdType.MESH,
            )

        k_rdmas = {(j, c): k_copy(j, c)
                   for j in BC_ORDER for c in range(NCH)}
        v_rdmas = {(j, b, c): v_copy(j, b, c)
                   for j in BC_ORDER for b in range(B) for c in range(NCH)}

        @pl.when(is_src)
        def _():
            k16[...] = k_ref[...].astype(jnp.bfloat16)
            for c in range(NCH):
                for j in BC_ORDER:
                    k_rdmas[(j, c)].start()
            v16[...] = v_ref[...].astype(jnp.bfloat16)
            k_loc[...] = k16[:, :, 0:HD]
            v_loc[...] = v16[:, :, 0:HD]
            for c in range(NCH):
                for j in BC_ORDER:
                    k_rdmas[(j, c)].wait_send()
            for b in range(B):
                for c in range(NCH):
                    for j in BC_ORDER:
                        v_rdmas[(j, b, c)].start()

        wq16 = wq_ref[...].astype(jnp.bfloat16)
        q = [
            jax.lax.dot(
                x_ref[b].astype(jnp.bfloat16), wq16,
                preferred_element_type=jnp.float32,
            )
            for b in range(B)
        ]
        wo16[...] = wo_ref[...].astype(jnp.bfloat16)

        qb = lax.broadcasted_iota(jnp.int32, (SQ, SKV), 0) // BLK
        kb = lax.broadcasted_iota(jnp.int32, (SQ, SKV), 1) // BLK
        mask = kb <= qb

        w16 = {}
        wsum = {}
        for c in range(NCH):
            @pl.when(not_src)
            def _(c=c):
                k_rdmas[(2, c)].wait_recv()

            for b in range(B):
                for h in (2 * c, 2 * c + 1):
                    qh = q[b][:, h * DH:(h + 1) * DH].astype(jnp.bfloat16)
                    kh = k_loc[b, :, h * DH:(h + 1) * DH]
                    s = jax.lax.dot_general(
                        qh, kh, (((1,), (1,)), ((), ())),
                        preferred_element_type=jnp.float32,
                    ) * 0.125
                    w = jnp.where(mask, jnp.exp(s), 0.0)
                    wsum[(b, h)] = jnp.sum(w, axis=1, keepdims=True)
                    w16[(b, h)] = w.astype(jnp.bfloat16)

        rs_rdmas = {}
        for b in range(B):
            @pl.when(not_src)
            def _(b=b):
                for c in range(NCH):
                    v_rdmas[(2, b, c)].wait_recv()

            ctx_h = []
            for h in range(HQ):
                vh = v_loc[b, :, h * DH:(h + 1) * DH]
                acc = jax.lax.dot(
                    w16[(b, h)], vh, preferred_element_type=jnp.float32,
                )
                ctx_h.append(acc / wsum[(b, h)])
            ctx = jnp.concatenate(ctx_h, axis=1).astype(jnp.bfloat16)
            p16[b] = jax.lax.dot(
                ctx, wo16[...], preferred_element_type=jnp.float32,
            ).astype(jnp.bfloat16)
            for d in (2, 1, 3):
                j = (my_i + d) % N_DEV
                r = pltpu.make_async_remote_copy(
                    src_ref=p16.at[b, :, pl.ds(QCOL * j, QCOL)],
                    dst_ref=rs_buf.at[d - 1, b],
                    send_sem=rs_send.at[d - 1, b],
                    recv_sem=rs_recv.at[d - 1, b],
                    device_id=(j,),
                    device_id_type=pl.DeviceIdType.MESH,
                )
                r.start()
                rs_rdmas[(d, b)] = r

        ag_rdmas = {}
        for b in range(B):
            q_red = p16[b, :, pl.ds(QCOL * my_i, QCOL)].astype(jnp.float32)
            for d in (1, 3, 2):
                rs_rdmas[(d, b)].wait_recv()
                q_red = q_red + rs_buf[d - 1, b].astype(jnp.float32)
            ag_full[b, :, pl.ds(QCOL * my_i, QCOL)] = q_red.astype(jnp.bfloat16)
            out_ref[b, :, pl.ds(QCOL * my_i, QCOL)] = q_red
            for d in (2, 1, 3):
                j = (my_i + d) % N_DEV
                r = pltpu.make_async_remote_copy(
                    src_ref=ag_full.at[b, :, pl.ds(QCOL * my_i, QCOL)],
                    dst_ref=ag_full.at[b, :, pl.ds(QCOL * my_i, QCOL)],
                    send_sem=ag_send.at[d - 1, b],
                    recv_sem=ag_recv.at[d - 1, b],
                    device_id=(j,),
                    device_id_type=pl.DeviceIdType.MESH,
                )
                r.start()
                ag_rdmas[(d, b)] = r

        for b in range(B):
            for d in (1, 3, 2):
                src = (my_i - d) % N_DEV
                ag_rdmas[(d, b)].wait_recv()
                out_ref[b, :, pl.ds(QCOL * src, QCOL)] = ag_full[
                    b, :, pl.ds(QCOL * src, QCOL)
                ].astype(jnp.float32)

        for r in rs_rdmas.values():
            r.wait_send()
        for r in ag_rdmas.values():
            r.wait_send()

        @pl.when(is_src)
        def _():
            for r in v_rdmas.values():
                r.wait_send()

    return pl.pallas_call(
        body,
        out_shape=jax.ShapeDtypeStruct((B, SQ, DMODEL), jnp.float32),
        in_specs=[pl.BlockSpec(memory_space=pltpu.VMEM)] * 5,
        out_specs=pl.BlockSpec(memory_space=pltpu.VMEM),
        scratch_shapes=[
            pltpu.VMEM((B, SKV, 16 * DH), jnp.bfloat16),
            pltpu.VMEM((B, SKV, 16 * DH), jnp.bfloat16),
            pltpu.VMEM((B, SKV, HD), jnp.bfloat16),
            pltpu.VMEM((B, SKV, HD), jnp.bfloat16),
            pltpu.VMEM((B, SQ, DMODEL), jnp.bfloat16),
            pltpu.VMEM((N_DEV - 1, B, SQ, QCOL), jnp.bfloat16),
            pltpu.VMEM((B, SQ, DMODEL), jnp.bfloat16),
            pltpu.VMEM((HD, DMODEL), jnp.bfloat16),
            pltpu.SemaphoreType.DMA((NCH, N_DEV - 1)),
            pltpu.SemaphoreType.DMA((NCH,)),
            pltpu.SemaphoreType.DMA((B, NCH, N_DEV - 1)),
            pltpu.SemaphoreType.DMA((B, NCH)),
            pltpu.SemaphoreType.DMA((N_DEV - 1, B)),
            pltpu.SemaphoreType.DMA((N_DEV - 1, B)),
            pltpu.SemaphoreType.DMA((N_DEV - 1, B)),
            pltpu.SemaphoreType.DMA((N_DEV - 1, B)),
        ],
        compiler_params=pltpu.CompilerParams(collective_id=0),
    )(x, Wq, k3, v3, Wo)
